# baseline (speedup 1.0000x reference)
import functools

import jax
import jax.numpy as jnp
from jax import lax
from jax.experimental import pallas as pl
from jax.experimental.pallas import tpu as pltpu
from jax.experimental.pallas import tpu_sc as plsc

B = 1024
D = 1024
E = 16
K = 2
H = 2048
DOUT = 1024
MED = 128
SH = 2048

TB = 128
NT = (B * K) // TB + E
NSLOT = NT * TB

NW = 32
TPW = B // NW

@functools.cache
def _sc_mesh():
    return plsc.VectorSubcoreMesh(core_axis_name="c", subcore_axis_name="s",
                                  num_cores=2, num_subcores=16)


def _worker_id():
    return lax.axis_index("s") * 2 + lax.axis_index("c")


def _sc_embed_body(ptab, mtab, pids, mids, x_out, th_out,
                   pidx, midx, xrows, trows, sem):
    base = _worker_id() * TPW
    pltpu.sync_copy(pids.at[pl.ds(base, TPW)], pidx)
    pltpu.async_copy(ptab.at[pidx], xrows, sem).wait()
    pltpu.sync_copy(xrows, x_out.at[pl.ds(base, TPW)])
    pltpu.sync_copy(mids.at[pl.ds(base, TPW)], midx)
    pltpu.async_copy(mtab.at[midx], trows, sem).wait()
    pltpu.sync_copy(trows, th_out.at[pl.ds(base, TPW)])


@functools.cache
def _sc_embed():
    return pl.kernel(
        _sc_embed_body,
        out_type=[jax.ShapeDtypeStruct((B, D), jnp.float32),
                  jax.ShapeDtypeStruct((B, MED), jnp.float32)],
        mesh=_sc_mesh(),
        scratch_types=[pltpu.VMEM((TPW,), jnp.int32),
                       pltpu.VMEM((TPW,), jnp.int32),
                       pltpu.VMEM((TPW, D), jnp.float32),
                       pltpu.VMEM((TPW, MED), jnp.float32),
                       pltpu.SemaphoreType.DMA],
    )


def _route_body(x_ref, gw_ref, gb_ref, w01_ref, slot01_ref, slotT_ref,
                xb_ref, te_ref):
    x = x_ref[...]
    xb_ref[...] = x.astype(jnp.bfloat16)
    gl = jnp.dot(x, gw_ref[...], preferred_element_type=jnp.float32) + gb_ref[...]
    ei = lax.broadcasted_iota(jnp.int32, (B, E), 1)

    m0 = jnp.max(gl, axis=1, keepdims=True)
    i0 = jnp.min(jnp.where(gl == m0, ei, E), axis=1, keepdims=True)
    gl2 = jnp.where(ei == i0, -jnp.inf, gl)
    m1 = jnp.max(gl2, axis=1, keepdims=True)
    i1 = jnp.min(jnp.where(gl2 == m1, ei, E), axis=1, keepdims=True)

    e1 = jnp.exp(m1 - m0)
    denom = 1.0 + e1
    w01_ref[...] = jnp.concatenate([1.0 / denom, e1 / denom], axis=1)

    m_top = (ei == i0).astype(jnp.float32)
    m_sec = (ei == i1).astype(jnp.float32)

    C = 256
    ri = lax.broadcasted_iota(jnp.int32, (C, C), 0)
    ci = lax.broadcasted_iota(jnp.int32, (C, C), 1)
    tril = (ci < ri).astype(jnp.float32)
    running = jnp.zeros((1, E), jnp.float32)
    ranks = []
    for m_full in (m_top, m_sec):
        rchunks = []
        for c in range(B // C):
            mc = m_full[c * C:(c + 1) * C, :]
            pref = jnp.dot(tril, mc, preferred_element_type=jnp.float32)
            rchunks.append(jnp.sum((pref + running) * mc, axis=1, keepdims=True))
            running = running + jnp.sum(mc, axis=0, keepdims=True)
        ranks.append(jnp.concatenate(rchunks, axis=0))

    counts = running.astype(jnp.int32)
    padded = ((counts + TB - 1) // TB) * TB
    ur = lax.broadcasted_iota(jnp.int32, (E, E), 0)
    uc = lax.broadcasted_iota(jnp.int32, (E, E), 1)
    upper = (ur < uc).astype(jnp.float32)
    offs = jnp.dot(padded.astype(jnp.float32), upper,
                   preferred_element_type=jnp.float32)
    slot0 = jnp.sum(offs * m_top, axis=1, keepdims=True) + ranks[0]
    slot1 = jnp.sum(offs * m_sec, axis=1, keepdims=True) + ranks[1]
    slot01 = jnp.concatenate([slot0, slot1], axis=1).astype(jnp.int32)
    slot01_ref[...] = slot01
    slotT_ref[...] = jnp.transpose(slot01, (1, 0))

    ends = offs + padded.astype(jnp.float32)
    tpos = lax.broadcasted_iota(jnp.int32, (NT, E), 0).astype(jnp.float32) * TB
    te = jnp.sum((tpos >= jnp.broadcast_to(ends, (NT, E))).astype(jnp.float32),
                 axis=1, keepdims=True)
    te = jnp.minimum(te, E - 1).astype(jnp.int32)
    te_ref[...] = jnp.broadcast_to(te, (NT, 128))


_route = pl.pallas_call(
    _route_body,
    out_shape=[jax.ShapeDtypeStruct((B, K), jnp.float32),
               jax.ShapeDtypeStruct((B, K), jnp.int32),
               jax.ShapeDtypeStruct((K, B), jnp.int32),
               jax.ShapeDtypeStruct((B, D), jnp.bfloat16),
               jax.ShapeDtypeStruct((NT, 128), jnp.int32)],
)


def _expert_body(te_ref, xb_ref, s01_ref, sT_ref, w01_ref,
                 w1_ref, b1_ref, w2_ref, b2_ref, routed_ref):
    t = pl.program_id(0)
    scol = t * TB + lax.broadcasted_iota(jnp.int32, (TB, 1), 0)
    srow = t * TB + lax.broadcasted_iota(jnp.int32, (1, TB), 1)

    g = ((sT_ref[0:1, :] == scol) | (sT_ref[1:2, :] == scol))
    xg = jnp.dot(g.astype(jnp.bfloat16), xb_ref[...],
                 preferred_element_type=jnp.float32)

    h = jnp.maximum(
        jnp.dot(xg.astype(jnp.bfloat16), w1_ref[0].astype(jnp.bfloat16),
                preferred_element_type=jnp.float32) + b1_ref[0], 0.0)
    y = (jnp.dot(h.astype(jnp.bfloat16), w2_ref[0].astype(jnp.bfloat16),
                 preferred_element_type=jnp.float32) + b2_ref[0])

    gtw = ((s01_ref[:, 0:1] == srow).astype(jnp.float32) * w01_ref[:, 0:1]
           + (s01_ref[:, 1:2] == srow).astype(jnp.float32) * w01_ref[:, 1:2])
    contrib = jnp.dot(gtw.astype(jnp.bfloat16), y.astype(jnp.bfloat16),
                      preferred_element_type=jnp.float32)

    @pl.when(t == 0)
    def _init():
        routed_ref[...] = contrib

    @pl.when(t > 0)
    def _acc():
        routed_ref[...] = routed_ref[...] + contrib


_experts = pl.pallas_call(
    _expert_body,
    grid_spec=pltpu.PrefetchScalarGridSpec(
        num_scalar_prefetch=1,
        grid=(NT,),
        in_specs=[
            pl.BlockSpec((B, D), lambda t, te: (0, 0)),
            pl.BlockSpec((B, K), lambda t, te: (0, 0)),
            pl.BlockSpec((K, B), lambda t, te: (0, 0)),
            pl.BlockSpec((B, K), lambda t, te: (0, 0)),
            pl.BlockSpec((1, D, H), lambda t, te: (te[t], 0, 0)),
            pl.BlockSpec((1, 1, H), lambda t, te: (te[t], 0, 0)),
            pl.BlockSpec((1, H, DOUT), lambda t, te: (te[t], 0, 0)),
            pl.BlockSpec((1, 1, DOUT), lambda t, te: (te[t], 0, 0)),
        ],
        out_specs=pl.BlockSpec((B, DOUT), lambda t, te: (0, 0)),
    ),
    out_shape=jax.ShapeDtypeStruct((B, DOUT), jnp.float32),
    compiler_params=pltpu.CompilerParams(
        dimension_semantics=("arbitrary",),
        vmem_limit_bytes=100 * 1024 * 1024,
    ),
)


RT = 256


def _final_body(xb_ref, w1_ref, b1_ref, w2_ref, b2_ref, rt_ref,
                th_ref, dw_ref, db_ref, cw_ref, cb_ref, out_ref):
    xb = xb_ref[...]
    hsh = jnp.maximum(
        jnp.dot(xb, w1_ref[...].astype(jnp.bfloat16),
                preferred_element_type=jnp.float32) + b1_ref[...], 0.0)
    sh = (jnp.dot(hsh.astype(jnp.bfloat16), w2_ref[...].astype(jnp.bfloat16),
                  preferred_element_type=jnp.float32) + b2_ref[...])

    hq = sh + rt_ref[...]
    bq = jnp.dot(hq, dw_ref[...], preferred_element_type=jnp.float32) + db_ref[...]
    aq = jnp.dot(hq, cw_ref[...], preferred_element_type=jnp.float32) + cb_ref[...]
    ability = jnp.sum(aq * th_ref[...], axis=1, keepdims=True)
    out_ref[...] = ability - bq


_final = pl.pallas_call(
    _final_body,
    grid=(B // RT,),
    in_specs=[
        pl.BlockSpec((RT, D), lambda i: (i, 0)),
        pl.BlockSpec((D, SH), lambda i: (0, 0)),
        pl.BlockSpec((1, SH), lambda i: (0, 0)),
        pl.BlockSpec((SH, DOUT), lambda i: (0, 0)),
        pl.BlockSpec((1, DOUT), lambda i: (0, 0)),
        pl.BlockSpec((RT, DOUT), lambda i: (i, 0)),
        pl.BlockSpec((RT, MED), lambda i: (i, 0)),
        pl.BlockSpec((DOUT, 1), lambda i: (0, 0)),
        pl.BlockSpec((1, 1), lambda i: (0, 0)),
        pl.BlockSpec((DOUT, MED), lambda i: (0, 0)),
        pl.BlockSpec((1, MED), lambda i: (0, 0)),
    ],
    out_specs=pl.BlockSpec((RT, 1), lambda i: (i, 0)),
    out_shape=jax.ShapeDtypeStruct((B, 1), jnp.float32),
    compiler_params=pltpu.CompilerParams(
        dimension_semantics=("arbitrary",),
        vmem_limit_bytes=100 * 1024 * 1024,
    ),
)


def kernel(params, prompt_table, model_ids, prompt_ids):
    p = params
    x, theta = _sc_embed()(prompt_table, p['model_table'],
                           prompt_ids.astype(jnp.int32),
                           model_ids.astype(jnp.int32))
    w01, slot01, slotT, xb, te2d = _route(x, p['gate_W'],
                                          p['gate_b'].reshape(1, E))
    routed = _experts(te2d[:, 0], xb, slot01, slotT, w01,
                      p['ex_W1'], p['ex_b1'].reshape(E, 1, H),
                      p['ex_W2'], p['ex_b2'].reshape(E, 1, DOUT))
    out = _final(xb, p['sh_W1'], p['sh_b1'].reshape(1, SH),
                 p['sh_W2'], p['sh_b2'].reshape(1, DOUT),
                 routed, theta,
                 p['diff_W'], p['diff_b'].reshape(1, 1),
                 p['disc_W'], p['disc_b'].reshape(1, MED))
    return out.reshape(B)


# speedup vs baseline: 1.0869x; 1.0869x over previous
import functools

import jax
import jax.numpy as jnp
from jax import lax
from jax.experimental import pallas as pl
from jax.experimental.pallas import tpu as pltpu
from jax.experimental.pallas import tpu_sc as plsc

B = 1024
D = 1024
E = 16
K = 2
H = 2048
DOUT = 1024
MED = 128
SH = 2048

TB = 128
NT = (B * K) // TB + E
NSLOT = NT * TB

NW = 32
TPW = B // NW

@functools.cache
def _sc_mesh():
    return plsc.VectorSubcoreMesh(core_axis_name="c", subcore_axis_name="s",
                                  num_cores=2, num_subcores=16)


def _worker_id():
    return lax.axis_index("s") * 2 + lax.axis_index("c")


def _sc_embed_body(ptab, mtab, pids, mids, x_out, th_out,
                   pidx, midx, xrows, trows, sem):
    base = _worker_id() * TPW
    pltpu.sync_copy(pids.at[pl.ds(base, TPW)], pidx)
    pltpu.async_copy(ptab.at[pidx], xrows, sem).wait()
    pltpu.sync_copy(xrows, x_out.at[pl.ds(base, TPW)])
    pltpu.sync_copy(mids.at[pl.ds(base, TPW)], midx)
    pltpu.async_copy(mtab.at[midx], trows, sem).wait()
    pltpu.sync_copy(trows, th_out.at[pl.ds(base, TPW)])


@functools.cache
def _sc_embed():
    return pl.kernel(
        _sc_embed_body,
        out_type=[jax.ShapeDtypeStruct((B, D), jnp.float32),
                  jax.ShapeDtypeStruct((B, MED), jnp.float32)],
        mesh=_sc_mesh(),
        scratch_types=[pltpu.VMEM((TPW,), jnp.int32),
                       pltpu.VMEM((TPW,), jnp.int32),
                       pltpu.VMEM((TPW, D), jnp.float32),
                       pltpu.VMEM((TPW, MED), jnp.float32),
                       pltpu.SemaphoreType.DMA],
    )


def _route_body(x_ref, gw_ref, gb_ref, w01_ref, slot01_ref, slotT_ref,
                xb_ref, te_ref, seg_ref, eseg_ref, ns_ref):
    x = x_ref[...]
    xb_ref[...] = x.astype(jnp.bfloat16)
    gl = jnp.dot(x, gw_ref[...], preferred_element_type=jnp.float32) + gb_ref[...]
    ei = lax.broadcasted_iota(jnp.int32, (B, E), 1)

    m0 = jnp.max(gl, axis=1, keepdims=True)
    i0 = jnp.min(jnp.where(gl == m0, ei, E), axis=1, keepdims=True)
    gl2 = jnp.where(ei == i0, -jnp.inf, gl)
    m1 = jnp.max(gl2, axis=1, keepdims=True)
    i1 = jnp.min(jnp.where(gl2 == m1, ei, E), axis=1, keepdims=True)

    e1 = jnp.exp(m1 - m0)
    denom = 1.0 + e1
    w01_ref[...] = jnp.concatenate([1.0 / denom, e1 / denom], axis=1)

    m_top = (ei == i0).astype(jnp.float32)
    m_sec = (ei == i1).astype(jnp.float32)

    C = 256
    ri = lax.broadcasted_iota(jnp.int32, (C, C), 0)
    ci = lax.broadcasted_iota(jnp.int32, (C, C), 1)
    tril = (ci < ri).astype(jnp.float32)
    running = jnp.zeros((1, E), jnp.float32)
    ranks = []
    for m_full in (m_top, m_sec):
        rchunks = []
        for c in range(B // C):
            mc = m_full[c * C:(c + 1) * C, :]
            pref = jnp.dot(tril, mc, preferred_element_type=jnp.float32)
            rchunks.append(jnp.sum((pref + running) * mc, axis=1, keepdims=True))
            running = running + jnp.sum(mc, axis=0, keepdims=True)
        ranks.append(jnp.concatenate(rchunks, axis=0))

    counts = running.astype(jnp.int32)
    padded = ((counts + TB - 1) // TB) * TB
    ur = lax.broadcasted_iota(jnp.int32, (E, E), 0)
    uc = lax.broadcasted_iota(jnp.int32, (E, E), 1)
    upper = (ur < uc).astype(jnp.float32)
    offs = jnp.dot(padded.astype(jnp.float32), upper,
                   preferred_element_type=jnp.float32)
    slot0 = jnp.sum(offs * m_top, axis=1, keepdims=True) + ranks[0]
    slot1 = jnp.sum(offs * m_sec, axis=1, keepdims=True) + ranks[1]
    slot01 = jnp.concatenate([slot0, slot1], axis=1).astype(jnp.int32)
    slot01_ref[...] = slot01
    slotT_ref[...] = jnp.transpose(slot01, (1, 0))

    ends = offs + padded.astype(jnp.float32)
    tpos = lax.broadcasted_iota(jnp.int32, (NT, E), 0).astype(jnp.float32) * TB
    te_f = jnp.sum((tpos >= jnp.broadcast_to(ends, (NT, E))).astype(jnp.float32),
                   axis=1, keepdims=True)
    te_f = jnp.minimum(te_f, E - 1)
    te_ref[...] = jnp.broadcast_to(te_f.astype(jnp.int32), (NT, 128))

    te_prev = jnp.concatenate([te_f[0:1], te_f[:-1]], axis=0)
    change = (te_f != te_prev).astype(jnp.float32)
    tr = lax.broadcasted_iota(jnp.int32, (NT, NT), 0)
    tc = lax.broadcasted_iota(jnp.int32, (NT, NT), 1)
    tril_inc = (tc <= tr).astype(jnp.float32)
    seg = jnp.dot(tril_inc, change, preferred_element_type=jnp.float32)
    nseg = seg[NT - 1:NT, 0:1] + 1.0
    firstt = jnp.concatenate([jnp.ones((1, 1), jnp.float32), change[1:]], axis=0)
    segT = jnp.transpose(seg, (1, 0))
    firstT = jnp.transpose(firstt, (1, 0))
    onehot = ((tr.astype(jnp.float32) == jnp.broadcast_to(segT, (NT, NT)))
              .astype(jnp.float32) * jnp.broadcast_to(firstT, (NT, NT)))
    eseg = jnp.dot(onehot, te_f, preferred_element_type=jnp.float32)
    seg_ref[...] = jnp.broadcast_to(seg.astype(jnp.int32), (NT, 128))
    eseg_ref[...] = jnp.broadcast_to(eseg.astype(jnp.int32), (NT, 128))
    ns_ref[...] = jnp.broadcast_to(nseg.astype(jnp.int32), (1, 128))


_route = pl.pallas_call(
    _route_body,
    out_shape=[jax.ShapeDtypeStruct((B, K), jnp.float32),
               jax.ShapeDtypeStruct((B, K), jnp.int32),
               jax.ShapeDtypeStruct((K, B), jnp.int32),
               jax.ShapeDtypeStruct((B, D), jnp.bfloat16),
               jax.ShapeDtypeStruct((NT, 128), jnp.int32),
               jax.ShapeDtypeStruct((NT, 128), jnp.int32),
               jax.ShapeDtypeStruct((NT, 128), jnp.int32),
               jax.ShapeDtypeStruct((1, 128), jnp.int32)],
)


NBUF = 3


def _expert_body(te_ref, seg_ref, eseg_ref, nseg_ref,
                 xb_ref, s01_ref, sT_ref, w01_ref,
                 w1_hbm, b1_ref, w2_hbm, b2_ref, routed_ref,
                 w1bufs, w2bufs, sems):
    t = pl.program_id(0)
    s = seg_ref[t]
    nseg = nseg_ref[0]

    def issue(si):
        slot = lax.rem(si, NBUF)
        e = eseg_ref[si]
        pltpu.make_async_copy(w1_hbm.at[e], w1bufs.at[slot],
                              sems.at[slot, 0]).start()
        pltpu.make_async_copy(w2_hbm.at[e], w2bufs.at[slot],
                              sems.at[slot, 1]).start()

    prev = seg_ref[jnp.maximum(t - 1, 0)]
    first = jnp.logical_or(t == 0, s != prev)

    @pl.when(t == 0)
    def _prime():
        for k in range(NBUF):
            @pl.when(k < nseg)
            def _():
                issue(k)

    @pl.when(jnp.logical_and(t > 0,
                             jnp.logical_and(first, s + (NBUF - 1) < nseg)))
    def _ahead():
        issue(s + (NBUF - 1))

    slot = lax.rem(s, NBUF)

    @pl.when(first)
    def _wait():
        e = eseg_ref[s]
        pltpu.make_async_copy(w1_hbm.at[e], w1bufs.at[slot],
                              sems.at[slot, 0]).wait()
        pltpu.make_async_copy(w2_hbm.at[e], w2bufs.at[slot],
                              sems.at[slot, 1]).wait()

    scol = t * TB + lax.broadcasted_iota(jnp.int32, (TB, 1), 0)
    srow = t * TB + lax.broadcasted_iota(jnp.int32, (1, TB), 1)

    g = ((sT_ref[0:1, :] == scol) | (sT_ref[1:2, :] == scol))
    xg = jnp.dot(g.astype(jnp.bfloat16), xb_ref[...],
                 preferred_element_type=jnp.float32)

    h = jnp.maximum(
        jnp.dot(xg.astype(jnp.bfloat16), w1bufs[slot].astype(jnp.bfloat16),
                preferred_element_type=jnp.float32) + b1_ref[0], 0.0)
    y = (jnp.dot(h.astype(jnp.bfloat16), w2bufs[slot].astype(jnp.bfloat16),
                 preferred_element_type=jnp.float32) + b2_ref[0])

    gtw = ((s01_ref[:, 0:1] == srow).astype(jnp.float32) * w01_ref[:, 0:1]
           + (s01_ref[:, 1:2] == srow).astype(jnp.float32) * w01_ref[:, 1:2])
    contrib = jnp.dot(gtw.astype(jnp.bfloat16), y.astype(jnp.bfloat16),
                      preferred_element_type=jnp.float32)

    @pl.when(t == 0)
    def _init():
        routed_ref[...] = contrib

    @pl.when(t > 0)
    def _acc():
        routed_ref[...] = routed_ref[...] + contrib


_experts = pl.pallas_call(
    _expert_body,
    grid_spec=pltpu.PrefetchScalarGridSpec(
        num_scalar_prefetch=4,
        grid=(NT,),
        in_specs=[
            pl.BlockSpec((B, D), lambda t, te, sg, es, ns: (0, 0)),
            pl.BlockSpec((B, K), lambda t, te, sg, es, ns: (0, 0)),
            pl.BlockSpec((K, B), lambda t, te, sg, es, ns: (0, 0)),
            pl.BlockSpec((B, K), lambda t, te, sg, es, ns: (0, 0)),
            pl.BlockSpec(memory_space=pl.ANY),
            pl.BlockSpec((1, 1, H), lambda t, te, sg, es, ns: (te[t], 0, 0)),
            pl.BlockSpec(memory_space=pl.ANY),
            pl.BlockSpec((1, 1, DOUT), lambda t, te, sg, es, ns: (te[t], 0, 0)),
        ],
        out_specs=pl.BlockSpec((B, DOUT), lambda t, te, sg, es, ns: (0, 0)),
        scratch_shapes=[
            pltpu.VMEM((NBUF, D, H), jnp.float32),
            pltpu.VMEM((NBUF, H, DOUT), jnp.float32),
            pltpu.SemaphoreType.DMA((NBUF, 2)),
        ],
    ),
    out_shape=jax.ShapeDtypeStruct((B, DOUT), jnp.float32),
    compiler_params=pltpu.CompilerParams(
        dimension_semantics=("arbitrary",),
        vmem_limit_bytes=110 * 1024 * 1024,
    ),
)


RT = 256


def _final_body(xb_ref, w1_ref, b1_ref, w2_ref, b2_ref, rt_ref,
                th_ref, dw_ref, db_ref, cw_ref, cb_ref, out_ref):
    xb = xb_ref[...]
    hsh = jnp.maximum(
        jnp.dot(xb, w1_ref[...].astype(jnp.bfloat16),
                preferred_element_type=jnp.float32) + b1_ref[...], 0.0)
    sh = (jnp.dot(hsh.astype(jnp.bfloat16), w2_ref[...].astype(jnp.bfloat16),
                  preferred_element_type=jnp.float32) + b2_ref[...])

    hq = sh + rt_ref[...]
    bq = jnp.dot(hq, dw_ref[...], preferred_element_type=jnp.float32) + db_ref[...]
    aq = jnp.dot(hq, cw_ref[...], preferred_element_type=jnp.float32) + cb_ref[...]
    ability = jnp.sum(aq * th_ref[...], axis=1, keepdims=True)
    out_ref[...] = ability - bq


_final = pl.pallas_call(
    _final_body,
    grid=(B // RT,),
    in_specs=[
        pl.BlockSpec((RT, D), lambda i: (i, 0)),
        pl.BlockSpec((D, SH), lambda i: (0, 0)),
        pl.BlockSpec((1, SH), lambda i: (0, 0)),
        pl.BlockSpec((SH, DOUT), lambda i: (0, 0)),
        pl.BlockSpec((1, DOUT), lambda i: (0, 0)),
        pl.BlockSpec((RT, DOUT), lambda i: (i, 0)),
        pl.BlockSpec((RT, MED), lambda i: (i, 0)),
        pl.BlockSpec((DOUT, 1), lambda i: (0, 0)),
        pl.BlockSpec((1, 1), lambda i: (0, 0)),
        pl.BlockSpec((DOUT, MED), lambda i: (0, 0)),
        pl.BlockSpec((1, MED), lambda i: (0, 0)),
    ],
    out_specs=pl.BlockSpec((RT, 1), lambda i: (i, 0)),
    out_shape=jax.ShapeDtypeStruct((B, 1), jnp.float32),
    compiler_params=pltpu.CompilerParams(
        dimension_semantics=("arbitrary",),
        vmem_limit_bytes=100 * 1024 * 1024,
    ),
)


def kernel(params, prompt_table, model_ids, prompt_ids):
    p = params
    x, theta = _sc_embed()(prompt_table, p['model_table'],
                           prompt_ids.astype(jnp.int32),
                           model_ids.astype(jnp.int32))
    (w01, slot01, slotT, xb, te2d, seg2d, eseg2d,
     ns2d) = _route(x, p['gate_W'], p['gate_b'].reshape(1, E))
    routed = _experts(te2d[:, 0], seg2d[:, 0], eseg2d[:, 0], ns2d[0:1, 0],
                      xb, slot01, slotT, w01,
                      p['ex_W1'], p['ex_b1'].reshape(E, 1, H),
                      p['ex_W2'], p['ex_b2'].reshape(E, 1, DOUT))
    out = _final(xb, p['sh_W1'], p['sh_b1'].reshape(1, SH),
                 p['sh_W2'], p['sh_b2'].reshape(1, DOUT),
                 routed, theta,
                 p['diff_W'], p['diff_b'].reshape(1, 1),
                 p['disc_W'], p['disc_b'].reshape(1, MED))
    return out.reshape(B)
